# baseline (speedup 1.0000x reference)
import functools

import jax
import jax.numpy as jnp
from jax import lax
from jax.experimental import pallas as pl
from jax.experimental.pallas import tpu as pltpu
from jax.experimental.pallas import tpu_sc as plsc

BATCH = 16384
EMBED = 32
HIDDEN = 64
ROLES = 16
PEDS = 8

NC = 2
NS = 16
NW = NC * NS
BPW = BATCH // NW


def _gather_body(cidx_hbm, sidx_hbm, ctab_hbm, stab_hbm,
                 cemb_hbm, semb_hbm,
                 cidx_v, sidx_v, crows_v, srows_v, sem_c, sem_s):
    wid = lax.axis_index("s") * NC + lax.axis_index("c")
    base = wid * BPW
    pltpu.sync_copy(cidx_hbm.at[pl.ds(base, BPW)], cidx_v)
    pltpu.sync_copy(sidx_hbm.at[pl.ds(base, BPW)], sidx_v)
    c_dma = pltpu.async_copy(ctab_hbm.at[cidx_v], crows_v, sem_c)
    s_dma = pltpu.async_copy(stab_hbm.at[sidx_v], srows_v, sem_s)
    c_dma.wait()
    pltpu.sync_copy(crows_v, cemb_hbm.at[pl.ds(base, BPW)])
    s_dma.wait()
    pltpu.sync_copy(srows_v, semb_hbm.at[pl.ds(base, BPW)])


@functools.cache
def _make_gather():
    return pl.kernel(
        _gather_body,
        out_type=(
            jax.ShapeDtypeStruct((BATCH, EMBED), jnp.float32),
            jax.ShapeDtypeStruct((BATCH, EMBED), jnp.float32),
        ),
        mesh=plsc.VectorSubcoreMesh(
            core_axis_name="c", subcore_axis_name="s",
            num_cores=NC, num_subcores=NS,
        ),
        scratch_types=[
            pltpu.VMEM((BPW,), jnp.int32),
            pltpu.VMEM((BPW,), jnp.int32),
            pltpu.VMEM((BPW, EMBED), jnp.float32),
            pltpu.VMEM((BPW, EMBED), jnp.float32),
            pltpu.SemaphoreType.DMA,
            pltpu.SemaphoreType.DMA,
        ],
        compiler_params=pltpu.CompilerParams(use_tc_tiling_on_sc=False),
    )


BLK = 2048


def _mlp_body(c_ref, s_ref, wh_ref, bh_ref, wr_ref, br_ref, wp_ref, bp_ref,
              role_ref, ped_ref):
    c = c_ref[...]
    s = s_ref[...]
    wh = wh_ref[...]
    h = jnp.dot(c, wh[:EMBED, :], preferred_element_type=jnp.float32)
    h = h + jnp.dot(s, wh[EMBED:, :], preferred_element_type=jnp.float32)
    h = jnp.maximum(h + bh_ref[...], 0.0)
    role_ref[...] = (
        jnp.dot(h, wr_ref[...], preferred_element_type=jnp.float32)
        + br_ref[...])
    ped_ref[...] = (
        jnp.dot(h, wp_ref[...], preferred_element_type=jnp.float32)
        + bp_ref[...])


def _mlp(cemb, semb, W_h, b_h2, W_r, b_r2, W_p, b_p2, interpret=False):
    rep = lambda shape: pl.BlockSpec(shape, lambda i: (0, 0))
    return pl.pallas_call(
        _mlp_body,
        grid=(BATCH // BLK,),
        in_specs=[
            pl.BlockSpec((BLK, EMBED), lambda i: (i, 0)),
            pl.BlockSpec((BLK, EMBED), lambda i: (i, 0)),
            rep((2 * EMBED, HIDDEN)),
            rep((1, HIDDEN)),
            rep((HIDDEN, ROLES)),
            rep((1, ROLES)),
            rep((HIDDEN, PEDS)),
            rep((1, PEDS)),
        ],
        out_specs=[
            pl.BlockSpec((BLK, ROLES), lambda i: (i, 0)),
            pl.BlockSpec((BLK, PEDS), lambda i: (i, 0)),
        ],
        out_shape=[
            jax.ShapeDtypeStruct((BATCH, ROLES), jnp.float32),
            jax.ShapeDtypeStruct((BATCH, PEDS), jnp.float32),
        ],
        interpret=interpret,
    )(cemb, semb, W_h, b_h2, W_r, b_r2, W_p, b_p2)


def kernel(concept_idx, style_idx, concept_table, style_table,
           W_h, b_h, W_r, b_r, W_p, b_p):
    cemb, semb = _make_gather()(concept_idx.astype(jnp.int32),
                         style_idx.astype(jnp.int32),
                         concept_table, style_table)
    role, ped = _mlp(cemb, semb, W_h, b_h.reshape(1, HIDDEN),
                     W_r, b_r.reshape(1, ROLES),
                     W_p, b_p.reshape(1, PEDS))
    return (role, ped)
